# Initial kernel scaffold; baseline (speedup 1.0000x reference)
import jax
import jax.numpy as jnp
from jax.experimental import pallas as pl


def kernel(x, forward_shuffle_idx):
    raise NotImplementedError("write your pallas kernel here")


# speedup vs baseline: 1.0524x; 1.0524x over previous
import jax
import jax.numpy as jnp
from jax.experimental import pallas as pl
from jax.experimental.pallas import tpu as pltpu


def _copy_body(idx_ref, x_ref, o_ref):
    o_ref[...] = x_ref[...]


def kernel(x, forward_shuffle_idx):
    B, C, H, W = x.shape
    HW = H * W
    x3 = x.reshape(B, C, 8, HW // 8)

    grid_spec = pltpu.PrefetchScalarGridSpec(
        num_scalar_prefetch=1,
        grid=(C,),
        in_specs=[
            pl.BlockSpec((B, 1, 8, HW // 8), lambda c, idx_ref: (0, idx_ref[c], 0, 0)),
        ],
        out_specs=pl.BlockSpec((B, 1, 8, HW // 8), lambda c, idx_ref: (0, c, 0, 0)),
    )
    out = pl.pallas_call(
        _copy_body,
        grid_spec=grid_spec,
        out_shape=jax.ShapeDtypeStruct(x3.shape, x.dtype),
    )(forward_shuffle_idx, x3)
    out = out.reshape(B, C, H, W)
    return (out, jnp.zeros((), x.dtype))
